# baseline (speedup 1.0000x reference)
import jax
import jax.numpy as jnp
from jax.experimental import pallas as pl
from jax.experimental.pallas import tpu as pltpu

R1 = 41
R2 = 41
TILE_H = 160


def _tile_sum(v):
    v = v.reshape(TILE_H // 8, 8, 640).sum(axis=0)
    return (v[:, 0:128] + v[:, 128:256] + v[:, 256:384]
            + v[:, 384:512] + v[:, 512:640])


def _loss_kernel(xy_ref, logits_ref, predx_ref, predy_ref, out_ref):
    bl = pl.program_id(0)
    t = pl.program_id(1)

    @pl.when(t == 0)
    def _():
        out_ref[...] = jnp.zeros_like(out_ref)

    n_bl = pl.num_programs(0)
    X = xy_ref[bl]
    Y = xy_ref[n_bl + bl]

    x = logits_ref[0, 0]
    px = predx_ref[0, 0]
    py = predy_ref[0, 0]

    row0 = (t * TILE_H).astype(jnp.float32)
    ii = row0 + jax.lax.broadcasted_iota(
        jnp.int32, (TILE_H, 640), 0).astype(jnp.float32)
    jj = jax.lax.broadcasted_iota(
        jnp.int32, (TILE_H, 640), 1).astype(jnp.float32)
    dx = X - ii
    dy = Y - jj
    inside = (dx * dx + dy * dy) <= float(R1 * R1)

    softplus = jnp.maximum(x, 0.0) + jnp.log1p(jnp.exp(-jnp.abs(x)))
    x_masked = jnp.where(inside, x, 0.0)
    inv_r2 = 1.0 / float(R2)
    l1x = jnp.where(inside, jnp.abs(px - dx * inv_r2), 0.0)
    l1y = jnp.where(inside, jnp.abs(py - dy * inv_r2), 0.0)
    cnt = jnp.where(inside, 1.0, 0.0)

    out_ref[0, 0] += _tile_sum(softplus)
    out_ref[0, 1] += _tile_sum(x_masked)
    out_ref[0, 2] += _tile_sum(l1x)
    out_ref[0, 3] += _tile_sum(l1y)
    out_ref[0, 4] += _tile_sum(cnt)


def kernel(featureMaps, landmarks):
    B, C, H, W = featureMaps.shape
    L = C // 3
    BL = B * L

    Xr = jnp.round(landmarks[:, :, 0] * (H - 1)).astype(jnp.float32)
    Yr = jnp.round(landmarks[:, :, 1] * (W - 1)).astype(jnp.float32)
    xy = jnp.concatenate([Xr.ravel(), Yr.ravel()])

    n_t = H // TILE_H

    def chan_spec(offset):
        return pl.BlockSpec(
            (1, 1, TILE_H, W),
            lambda bl, t, xy_ref, offset=offset: (bl // L, offset + bl % L, t, 0),
        )

    grid_spec = pltpu.PrefetchScalarGridSpec(
        num_scalar_prefetch=1,
        grid=(BL, n_t),
        in_specs=[chan_spec(0), chan_spec(L), chan_spec(2 * L)],
        out_specs=pl.BlockSpec((1, 5, 8, 128), lambda bl, t, xy_ref: (bl, 0, 0, 0)),
    )

    partials = pl.pallas_call(
        _loss_kernel,
        out_shape=jax.ShapeDtypeStruct((BL, 5, 8, 128), jnp.float32),
        grid_spec=grid_spec,
        compiler_params=pltpu.CompilerParams(
            dimension_semantics=("parallel", "arbitrary"),
        ),
        name="fusion_loss",
    )(xy, featureMaps, featureMaps, featureMaps)

    sums = jnp.sum(partials, axis=(2, 3))
    sp, xm, l1x, l1y, cnt = (sums[:, k] for k in range(5))
    bce = 2.0 * (sp - xm) / float(H * W)
    l1 = (l1x + l1y) / cnt
    return jnp.mean(bce + l1)


# speedup vs baseline: 1.0907x; 1.0907x over previous
import jax
import jax.numpy as jnp
from jax.experimental import pallas as pl
from jax.experimental.pallas import tpu as pltpu

R1 = 41
R2 = 41
TILE_H = 160
LOG2E = 1.4426950408889634
LN2 = 0.6931471805599453


def _tile_sum(v):
    acc = v[0:8]
    for k in range(1, TILE_H // 8):
        acc = acc + v[8 * k:8 * k + 8]
    return (acc[:, 0:128] + acc[:, 128:256] + acc[:, 256:384]
            + acc[:, 384:512] + acc[:, 512:640])


def kernel(featureMaps, landmarks):
    B, C, H, W = featureMaps.shape
    L = C // 3
    BL = B * L
    n_t = H // TILE_H

    Xi = jnp.round(landmarks[:, :, 0] * (H - 1)).astype(jnp.int32).ravel()
    Yi = jnp.round(landmarks[:, :, 1] * (W - 1)).astype(jnp.int32).ravel()
    scalars = jnp.concatenate([Xi, Yi])

    mesh = pltpu.create_tensorcore_mesh("core")
    out_init = jnp.zeros((BL, 4, 8, 128), jnp.float32)

    def state_fn(refs):
        fm_ref, sc_ref, out_ref = refs

        @pl.core_map(mesh)
        def _():
            def scoped(sc_smem, sem):
                cp = pltpu.make_async_copy(sc_ref, sc_smem, sem)
                cp.start()
                cp.wait()

                def inner(indices, logits_ref, predx_ref, predy_ref, acc_ref):
                    bl, t = indices

                    @pl.when(t == 0)
                    def _():
                        acc_ref[...] = jnp.zeros_like(acc_ref)

                    X = sc_smem[bl].astype(jnp.float32)
                    Y = sc_smem[BL + bl].astype(jnp.float32)

                    x = logits_ref[0, 0]
                    px = predx_ref[0, 0]
                    py = predy_ref[0, 0]

                    row0 = (t * TILE_H).astype(jnp.float32)
                    ii = row0 + jax.lax.broadcasted_iota(
                        jnp.int32, (TILE_H, 640), 0).astype(jnp.float32)
                    jj = jax.lax.broadcasted_iota(
                        jnp.int32, (TILE_H, 640), 1).astype(jnp.float32)
                    dx = X - ii
                    dy = Y - jj
                    inside = (dx * dx + dy * dy) <= float(R1 * R1)

                    sp2 = jnp.log2(1.0 + jnp.exp2(x * LOG2E))
                    x_masked = jnp.where(inside, x, 0.0)
                    inv_r2 = 1.0 / float(R2)
                    l1 = jnp.where(
                        inside,
                        jnp.abs(px - dx * inv_r2) + jnp.abs(py - dy * inv_r2),
                        0.0)
                    cnt = jnp.where(inside, 1.0, 0.0)

                    acc_ref[0, 0] += _tile_sum(sp2)
                    acc_ref[0, 1] += _tile_sum(x_masked)
                    acc_ref[0, 2] += _tile_sum(l1)
                    acc_ref[0, 3] += _tile_sum(cnt)

                pltpu.emit_pipeline(
                    inner,
                    grid=(BL, n_t),
                    in_specs=[
                        pl.BlockSpec((1, 1, TILE_H, W),
                                     lambda bl, t: (bl // L, bl % L, t, 0)),
                        pl.BlockSpec((1, 1, TILE_H, W),
                                     lambda bl, t: (bl // L, L + bl % L, t, 0)),
                        pl.BlockSpec((1, 1, TILE_H, W),
                                     lambda bl, t: (bl // L, 2 * L + bl % L, t, 0)),
                    ],
                    out_specs=[
                        pl.BlockSpec((1, 4, 8, 128),
                                     lambda bl, t: (bl, 0, 0, 0)),
                    ],
                    core_axis_name="core",
                    dimension_semantics=(
                        pltpu.GridDimensionSemantics.PARALLEL,
                        pltpu.GridDimensionSemantics.ARBITRARY,
                    ),
                    _explicit_indices=True,
                )(fm_ref, fm_ref, fm_ref, out_ref)

            pl.run_scoped(
                scoped,
                pltpu.SMEM((2 * BL,), jnp.int32),
                pltpu.SemaphoreType.DMA,
            )

    _, _, partials = pl.run_state(state_fn)(
        (featureMaps, scalars, out_init))

    sums = jnp.sum(partials, axis=(2, 3))
    sp = sums[:, 0] * LN2
    xm = sums[:, 1]
    l1 = sums[:, 2]
    cnt = sums[:, 3]
    bce = 2.0 * (sp - xm) / float(H * W)
    return jnp.mean(bce + l1 / cnt)
